# baseline (speedup 1.0000x reference)
import functools

import jax
import jax.numpy as jnp
from jax import lax
from jax.experimental import pallas as pl
from jax.experimental.pallas import tpu as pltpu
from jax.experimental.pallas import tpu_sc as plsc

VOCAB = 100000
D = 2048
BATCH = 4
SEQ = 4096
B_TOT = BATCH * SEQ

N_SC = 12288
N_TC = B_TOT - N_SC
G = 16

NC = 2
NS = 16
NW = NC * NS
BPW = N_SC // NW

C = 8
NCHUNK = BPW // C
R = 4
L = 2


def _emb_body(idx_hbm, table_hbm, out_hbm, idx_v, rows_v, gsem, ssem):
    wid = lax.axis_index("s") * NC + lax.axis_index("c")
    base = wid * BPW

    pltpu.sync_copy(idx_hbm.at[wid], idx_v)

    def _gather_start(c, b):
        pltpu.async_copy(table_hbm.at[idx_v.at[c]], rows_v.at[b], gsem.at[b])

    def _gather_wait(c, b):
        pltpu.make_async_copy(
            table_hbm.at[idx_v.at[c]], rows_v.at[b], gsem.at[b]
        ).wait()

    def _scatter_start(c, b):
        pltpu.async_copy(
            rows_v.at[b], out_hbm.at[pl.ds(base + c * C, C)], ssem.at[b]
        )

    def _scatter_wait(c, b):
        pltpu.make_async_copy(
            rows_v.at[b], out_hbm.at[pl.ds(base + c * C, C)], ssem.at[b]
        ).wait()

    for g in range(L):
        _gather_start(g, g)

    for c in range(R):
        _gather_wait(c, c % R)
        _scatter_start(c, c % R)
        g = c + L
        bg = g % R
        if g - R >= 0:
            _scatter_wait(g - R, bg)
        _gather_start(g, bg)

    @pl.loop(R, NCHUNK - R, step=R)
    def _steady(i):
        for b in range(R):
            c = i + b
            _gather_wait(c, b)
            _scatter_start(c, b)
            bg = (b + L) % R
            _scatter_wait(c + L - R, bg)
            _gather_start(c + L, bg)

    for c in range(NCHUNK - R, NCHUNK):
        b = c % R
        _gather_wait(c, b)
        _scatter_start(c, b)
        g = c + L
        if g < NCHUNK:
            bg = g % R
            _scatter_wait(g - R, bg)
            _gather_start(g, bg)

    for c in range(NCHUNK - R, NCHUNK):
        _scatter_wait(c, c % R)


def _tc_body(ids_ref, *refs):
    del ids_ref
    in_refs = refs[:G]
    out_ref = refs[G]
    for j in range(G):
        out_ref[j, :] = in_refs[j][0, 0, :]


@jax.jit
def _emb(sc_ids, tc_ids, table):
    mesh = plsc.VectorSubcoreMesh(core_axis_name="c", subcore_axis_name="s")
    run = pl.kernel(
        _emb_body,
        out_type=jax.ShapeDtypeStruct((N_SC, D), jnp.float32),
        mesh=mesh,
        scratch_types=[
            pltpu.VMEM((NCHUNK, C), jnp.int32),
            pltpu.VMEM((R, C, D), jnp.float32),
            pltpu.SemaphoreType.DMA((R,)),
            pltpu.SemaphoreType.DMA((R,)),
        ],
    )
    sc_out = run(sc_ids, table)

    grid_spec = pltpu.PrefetchScalarGridSpec(
        num_scalar_prefetch=1,
        grid=(N_TC // G,),
        in_specs=[
            pl.BlockSpec((1, 1, D), (lambda i, ids, j=j: (ids[i * G + j], 0, 0)))
            for j in range(G)
        ],
        out_specs=pl.BlockSpec((G, D), lambda i, ids: (i, 0)),
    )
    table3 = table.reshape(VOCAB, 1, D)
    tc_out = pl.pallas_call(
        _tc_body,
        grid_spec=grid_spec,
        out_shape=jax.ShapeDtypeStruct((N_TC, D), jnp.float32),
    )(tc_ids, *([table3] * G))
    return jnp.concatenate([sc_out, tc_out], axis=0)


def kernel(input_ids, word_embeddings):
    flat_ids = input_ids.reshape(-1).astype(jnp.int32)
    sc_ids = flat_ids[:N_SC].reshape(NW, NCHUNK, C)
    tc_ids = flat_ids[N_SC:]
    out = _emb(sc_ids, tc_ids, word_embeddings)
    return out.reshape(input_ids.shape[0], input_ids.shape[1], D)


# speedup vs baseline: 8.3292x; 8.3292x over previous
import functools

import jax
import jax.numpy as jnp
from jax import lax
from jax.experimental import pallas as pl
from jax.experimental.pallas import tpu as pltpu
from jax.experimental.pallas import tpu_sc as plsc

VOCAB = 100000
D = 2048
BATCH = 4
SEQ = 4096
B_TOT = BATCH * SEQ

NC = 2
NS = 16
NW = NC * NS
BPW = B_TOT // NW

C = 8
NCHUNK = BPW // C
R = 4
L = 2


def _emb_body(idx_hbm, table_hbm, out_hbm, idx_v, rows_v, gsem, ssem):
    wid = lax.axis_index("s") * NC + lax.axis_index("c")
    base = wid * BPW

    WPB = SEQ // BPW
    pltpu.sync_copy(idx_hbm.at[wid // WPB, pl.ds((wid % WPB) * BPW, BPW)], idx_v)

    def _gather_start(c, b):
        pltpu.async_copy(
            table_hbm.at[idx_v.at[pl.ds(c * C, C)]], rows_v.at[b], gsem.at[b]
        )

    def _gather_wait(c, b):
        pltpu.make_async_copy(
            table_hbm.at[idx_v.at[pl.ds(c * C, C)]], rows_v.at[b], gsem.at[b]
        ).wait()

    def _scatter_start(c, b):
        pltpu.async_copy(
            rows_v.at[b], out_hbm.at[pl.ds(base + c * C, C)], ssem.at[b]
        )

    def _scatter_wait(c, b):
        pltpu.make_async_copy(
            rows_v.at[b], out_hbm.at[pl.ds(base + c * C, C)], ssem.at[b]
        ).wait()

    for g in range(L):
        _gather_start(g, g)

    for c in range(R):
        _gather_wait(c, c % R)
        _scatter_start(c, c % R)
        g = c + L
        bg = g % R
        if g - R >= 0:
            _scatter_wait(g - R, bg)
        _gather_start(g, bg)

    @pl.loop(R, NCHUNK - R, step=R)
    def _steady(i):
        for b in range(R):
            c = i + b
            _gather_wait(c, b)
            _scatter_start(c, b)
            bg = (b + L) % R
            _scatter_wait(c + L - R, bg)
            _gather_start(c + L, bg)

    for c in range(NCHUNK - R, NCHUNK):
        b = c % R
        _gather_wait(c, b)
        _scatter_start(c, b)
        g = c + L
        if g < NCHUNK:
            bg = g % R
            _scatter_wait(g - R, bg)
            _gather_start(g, bg)

    for c in range(NCHUNK - R, NCHUNK):
        _scatter_wait(c, c % R)


@jax.jit
def _emb(flat_ids, table):
    mesh = plsc.VectorSubcoreMesh(core_axis_name="c", subcore_axis_name="s")
    run = pl.kernel(
        _emb_body,
        out_type=jax.ShapeDtypeStruct((B_TOT, D), jnp.float32),
        mesh=mesh,
        scratch_types=[
            pltpu.VMEM((BPW,), jnp.int32),
            pltpu.VMEM((R, C, D), jnp.float32),
            pltpu.SemaphoreType.DMA((R,)),
            pltpu.SemaphoreType.DMA((R,)),
        ],
    )
    return run(flat_ids, table)


def kernel(input_ids, word_embeddings):
    out = _emb(input_ids.astype(jnp.int32), word_embeddings)
    return out.reshape(input_ids.shape[0], input_ids.shape[1], D)
